# Initial kernel scaffold; baseline (speedup 1.0000x reference)
import jax
import jax.numpy as jnp
from jax.experimental import pallas as pl


def kernel(x, HOURE):
    raise NotImplementedError("write your pallas kernel here")


# speedup vs baseline: 1.5932x; 1.5932x over previous
import functools

import jax
import jax.numpy as jnp
from jax import lax
from jax.experimental import pallas as pl
from jax.experimental.pallas import tpu as pltpu
from jax.experimental.pallas import tpu_sc as plsc

_G = 128
_CH = 4


def _sc_gather(idx_flat, table):
    n_idx = idx_flat.shape[0]
    d = table.shape[1]
    nc, ns = 2, 16
    nw = nc * ns
    per_w = n_idx // nw
    chunk = _CH * _G
    n_chunks = per_w // chunk
    mesh = plsc.VectorSubcoreMesh(core_axis_name="c", subcore_axis_name="s")

    @functools.partial(
        pl.kernel,
        mesh=mesh,
        out_type=jax.ShapeDtypeStruct((n_idx, d), jnp.float32),
        scratch_types=[
            pltpu.VMEM((chunk,), jnp.int32),
            pltpu.VMEM((chunk, d), jnp.float32),
            pltpu.SemaphoreType.DMA,
        ],
        compiler_params=pltpu.CompilerParams(use_tc_tiling_on_sc=False),
    )
    def k(idx_hbm, tab_hbm, out_hbm, idx_v, rows_v, sem):
        wid = lax.axis_index("s") * nc + lax.axis_index("c")
        base0 = wid * per_w

        def body(i, carry):
            base = base0 + i * chunk
            pltpu.sync_copy(idx_hbm.at[pl.ds(base, chunk)], idx_v)
            copies = [
                pltpu.async_copy(
                    tab_hbm.at[idx_v.at[pl.ds(j * _G, _G)]],
                    rows_v.at[pl.ds(j * _G, _G)],
                    sem,
                )
                for j in range(_CH)
            ]
            for c in copies:
                c.wait()
            pltpu.sync_copy(rows_v, out_hbm.at[pl.ds(base, chunk)])
            return carry

        lax.fori_loop(0, n_chunks, body, 0)

    return k(idx_flat, table)


def kernel(x, HOURE):
    b0, b1 = x.shape
    total = b0 * b1
    d = HOURE.shape[1]
    out = _sc_gather(x.reshape(total), HOURE)
    return out.reshape(b0, b1, d)


# speedup vs baseline: 1.5976x; 1.0028x over previous
import functools

import jax
import jax.numpy as jnp
from jax import lax
from jax.experimental import pallas as pl
from jax.experimental.pallas import tpu as pltpu
from jax.experimental.pallas import tpu_sc as plsc

_CHUNK = 512
_NBUF = 2


def _sc_gather(idx_flat, table):
    n_idx = idx_flat.shape[0]
    d = table.shape[1]
    nc, ns = 2, 16
    nw = nc * ns
    per_w = n_idx // nw
    n_chunks = per_w // _CHUNK
    mesh = plsc.VectorSubcoreMesh(core_axis_name="c", subcore_axis_name="s")

    @functools.partial(
        pl.kernel,
        mesh=mesh,
        out_type=jax.ShapeDtypeStruct((n_idx, d), jnp.float32),
        scratch_types=[
            pltpu.VMEM((_NBUF, _CHUNK), jnp.int32),
            pltpu.VMEM((_NBUF, _CHUNK, d), jnp.float32),
            [pltpu.SemaphoreType.DMA] * _NBUF,
            [pltpu.SemaphoreType.DMA] * _NBUF,
        ],
        compiler_params=pltpu.CompilerParams(use_tc_tiling_on_sc=False),
    )
    def k(idx_hbm, tab_hbm, out_hbm, idx_v, rows_v, gsems, wsems):
        wid = lax.axis_index("s") * nc + lax.axis_index("c")
        base0 = wid * per_w

        def fire(i, b):
            base = base0 + i * _CHUNK
            pltpu.sync_copy(idx_hbm.at[pl.ds(base, _CHUNK)], idx_v.at[b])
            pltpu.async_copy(
                tab_hbm.at[idx_v.at[b]], rows_v.at[b], gsems[b]
            )

        def drain_and_write(i, b):
            base = base0 + i * _CHUNK
            pltpu.make_async_copy(
                tab_hbm.at[idx_v.at[b]], rows_v.at[b], gsems[b]
            ).wait()
            pltpu.async_copy(
                rows_v.at[b], out_hbm.at[pl.ds(base, _CHUNK)], wsems[b]
            )

        def wait_write(i, b):
            base = base0 + i * _CHUNK
            pltpu.make_async_copy(
                rows_v.at[b], out_hbm.at[pl.ds(base, _CHUNK)], wsems[b]
            ).wait()

        fire(0, 0)
        for i in range(n_chunks):
            b = i % _NBUF
            nxt = (i + 1) % _NBUF
            if i + 1 < n_chunks:
                if i >= 1:
                    wait_write(i - 1, nxt)
                fire(i + 1, nxt)
            drain_and_write(i, b)
        wait_write(n_chunks - 2, (n_chunks - 2) % _NBUF)
        wait_write(n_chunks - 1, (n_chunks - 1) % _NBUF)

    return k(idx_flat, table)


def kernel(x, HOURE):
    b0, b1 = x.shape
    total = b0 * b1
    d = HOURE.shape[1]
    out = _sc_gather(x.reshape(total), HOURE)
    return out.reshape(b0, b1, d)


# speedup vs baseline: 3.4211x; 2.1414x over previous
import functools

import jax
import jax.numpy as jnp
from jax import lax
from jax.experimental import pallas as pl
from jax.experimental.pallas import tpu as pltpu
from jax.experimental.pallas import tpu_sc as plsc

_CHUNK = 256
_NBUF = 2


def _sc_gather(idx_flat, table):
    n_idx = idx_flat.shape[0]
    d = table.shape[1]
    nc, ns = 2, 16
    nw = nc * ns
    per_w = n_idx // nw
    n_chunks = per_w // _CHUNK
    mesh = plsc.VectorSubcoreMesh(core_axis_name="c", subcore_axis_name="s")

    @functools.partial(
        pl.kernel,
        mesh=mesh,
        out_type=jax.ShapeDtypeStruct((n_idx, d), jnp.float32),
        scratch_types=[
            pltpu.VMEM((_NBUF, _CHUNK), jnp.int32),
            pltpu.VMEM((_NBUF, _CHUNK, d), jnp.float32),
            [pltpu.SemaphoreType.DMA] * _NBUF,
            [pltpu.SemaphoreType.DMA] * _NBUF,
        ],
        compiler_params=pltpu.CompilerParams(use_tc_tiling_on_sc=False),
    )
    def k(idx_hbm, tab_hbm, out_hbm, idx_v, rows_v, gsems, wsems):
        wid = lax.axis_index("s") * nc + lax.axis_index("c")
        base0 = wid * per_w

        def fire(i, b):
            base = base0 + i * _CHUNK
            pltpu.sync_copy(idx_hbm.at[pl.ds(base, _CHUNK)], idx_v.at[b])
            pltpu.async_copy(
                tab_hbm.at[idx_v.at[b]], rows_v.at[b], gsems[b]
            )

        def drain_and_write(i, b):
            base = base0 + i * _CHUNK
            pltpu.make_async_copy(
                tab_hbm.at[idx_v.at[b]], rows_v.at[b], gsems[b]
            ).wait()
            pltpu.async_copy(
                rows_v.at[b], out_hbm.at[pl.ds(base, _CHUNK)], wsems[b]
            )

        def wait_write(i, b):
            base = base0 + i * _CHUNK
            pltpu.make_async_copy(
                rows_v.at[b], out_hbm.at[pl.ds(base, _CHUNK)], wsems[b]
            ).wait()

        fire(0, 0)
        for i in range(n_chunks):
            b = i % _NBUF
            nxt = (i + 1) % _NBUF
            if i + 1 < n_chunks:
                if i >= 1:
                    wait_write(i - 1, nxt)
                fire(i + 1, nxt)
            drain_and_write(i, b)
        wait_write(n_chunks - 2, (n_chunks - 2) % _NBUF)
        wait_write(n_chunks - 1, (n_chunks - 1) % _NBUF)

    return k(idx_flat, table)


def kernel(x, HOURE):
    b0, b1 = x.shape
    total = b0 * b1
    v, d = HOURE.shape
    xf = x.reshape(total // 2, 2)
    pidx = xf[:, 0] * v + xf[:, 1]
    tab2 = jnp.concatenate(
        [
            jnp.broadcast_to(HOURE[:, None, :], (v, v, d)),
            jnp.broadcast_to(HOURE[None, :, :], (v, v, d)),
        ],
        axis=-1,
    ).reshape(v * v, 2 * d)
    out = _sc_gather(pidx, tab2)
    return out.reshape(b0, b1, d)
